# baseline (speedup 1.0000x reference)
import functools

import jax
import jax.numpy as jnp
from jax.experimental import pallas as pl
from jax.experimental.pallas import tpu as pltpu
from jax.experimental.pallas import tpu_sc as plsc

_BR = 8
_LANES = 16


def _sc_add(xf, pe):
    R, D = xf.shape
    S = pe.shape[0]
    mesh = plsc.VectorSubcoreMesh(core_axis_name="c", subcore_axis_name="s")

    @functools.partial(
        pl.kernel,
        out_type=jax.ShapeDtypeStruct((R, D), xf.dtype),
        mesh=mesh,
    )
    def k(x_hbm, pe_hbm, o_hbm):
        def body(x_vmem, pe_vmem, o_vmem):
            @pl.loop(0, _BR)
            def _row(r):
                @pl.loop(0, D, step=_LANES)
                def _col(c):
                    slc = (pl.ds(r, 1), pl.ds(c, _LANES))
                    o_vmem.at[*slc][...] = (
                        x_vmem.at[*slc][...] + pe_vmem.at[*slc][...]
                    )

        pltpu.emit_pipeline(
            body,
            grid=(R // _BR, 1),
            in_specs=[
                pl.BlockSpec((_BR, D), lambda i, j: (i, j)),
                pl.BlockSpec((_BR, D), lambda i, j: (i % (S // _BR), j)),
            ],
            out_specs=[pl.BlockSpec((_BR, D), lambda i, j: (i, j))],
            core_axis_name=("c", "s"),
            dimension_semantics=(pltpu.PARALLEL, pltpu.PARALLEL),
        )(x_hbm, pe_hbm, o_hbm)

    return k(xf, pe)


def kernel(x, pe_weight):
    B, S, D = x.shape
    pe = pe_weight[:S]
    out = _sc_add(x.reshape(B * S, D), pe)
    return out.reshape(B, S, D)


# speedup vs baseline: 3.0131x; 3.0131x over previous
import functools

import jax
import jax.numpy as jnp
from jax.experimental import pallas as pl
from jax.experimental.pallas import tpu as pltpu
from jax.experimental.pallas import tpu_sc as plsc

_BR = 8
_LANES = 16


def _sc_add(xf, pe):
    R, D = xf.shape
    S = pe.shape[0]
    mesh = plsc.VectorSubcoreMesh(core_axis_name="c", subcore_axis_name="s")

    @functools.partial(
        pl.kernel,
        out_type=jax.ShapeDtypeStruct((R, D), xf.dtype),
        mesh=mesh,
    )
    def k(x_hbm, pe_hbm, o_hbm):
        def body(x_vmem, pe_vmem, o_vmem):
            @pl.loop(0, _BR)
            def _row(r):
                @pl.loop(0, D, step=_LANES)
                def _col(c):
                    slc = (pl.ds(r, 1), pl.ds(c, _LANES))
                    o_vmem.at[*slc][...] = (
                        x_vmem.at[*slc][...] + pe_vmem.at[*slc][...]
                    )

        pltpu.emit_pipeline(
            body,
            grid=(R // _BR, 1),
            in_specs=[
                pl.BlockSpec((_BR, D), lambda i, j: (i, j)),
                pl.BlockSpec((_BR, D), lambda i, j: (i % (S // _BR), j)),
            ],
            out_specs=[pl.BlockSpec((_BR, D), lambda i, j: (i, j))],
            core_axis_name=("c", "s"),
            dimension_semantics=(pltpu.PARALLEL, pltpu.PARALLEL),
        )(x_hbm, pe_hbm, o_hbm)

    return k(xf, pe)


_BS = 512


def _tc_body(x_ref, pe_ref, o_ref):
    o_ref[...] = x_ref[...] + pe_ref[...]


def _tc_add(x, pe):
    B, S, D = x.shape
    grid = (S // _BS, B)
    return pl.pallas_call(
        _tc_body,
        grid=grid,
        in_specs=[
            pl.BlockSpec((1, _BS, D), lambda i, b: (b, i, 0)),
            pl.BlockSpec((_BS, D), lambda i, b: (i, 0)),
        ],
        out_specs=pl.BlockSpec((1, _BS, D), lambda i, b: (b, i, 0)),
        out_shape=jax.ShapeDtypeStruct((B, S, D), x.dtype),
        compiler_params=pltpu.CompilerParams(
            dimension_semantics=("parallel", "parallel"),
        ),
    )(x, pe)


def kernel(x, pe_weight):
    B, S, D = x.shape
    pe = pe_weight[:S]
    return _tc_add(x, pe)


# speedup vs baseline: 3.2753x; 1.0870x over previous
import functools

import jax
import jax.numpy as jnp
from jax.experimental import pallas as pl
from jax.experimental.pallas import tpu as pltpu
from jax.experimental.pallas import tpu_sc as plsc

_BR = 8
_LANES = 16


def _sc_add(xf, pe):
    R, D = xf.shape
    S = pe.shape[0]
    mesh = plsc.VectorSubcoreMesh(core_axis_name="c", subcore_axis_name="s")

    @functools.partial(
        pl.kernel,
        out_type=jax.ShapeDtypeStruct((R, D), xf.dtype),
        mesh=mesh,
    )
    def k(x_hbm, pe_hbm, o_hbm):
        def body(x_vmem, pe_vmem, o_vmem):
            @pl.loop(0, _BR)
            def _row(r):
                @pl.loop(0, D, step=_LANES)
                def _col(c):
                    slc = (pl.ds(r, 1), pl.ds(c, _LANES))
                    o_vmem.at[*slc][...] = (
                        x_vmem.at[*slc][...] + pe_vmem.at[*slc][...]
                    )

        pltpu.emit_pipeline(
            body,
            grid=(R // _BR, 1),
            in_specs=[
                pl.BlockSpec((_BR, D), lambda i, j: (i, j)),
                pl.BlockSpec((_BR, D), lambda i, j: (i % (S // _BR), j)),
            ],
            out_specs=[pl.BlockSpec((_BR, D), lambda i, j: (i, j))],
            core_axis_name=("c", "s"),
            dimension_semantics=(pltpu.PARALLEL, pltpu.PARALLEL),
        )(x_hbm, pe_hbm, o_hbm)

    return k(xf, pe)


_BS = 1024


def _tc_body(x_ref, pe_ref, o_ref):
    o_ref[...] = x_ref[...] + pe_ref[...]


def _tc_add(x, pe):
    B, S, D = x.shape
    grid = (S // _BS, B)
    return pl.pallas_call(
        _tc_body,
        grid=grid,
        in_specs=[
            pl.BlockSpec((1, _BS, D), lambda i, b: (b, i, 0)),
            pl.BlockSpec((_BS, D), lambda i, b: (i, 0)),
        ],
        out_specs=pl.BlockSpec((1, _BS, D), lambda i, b: (b, i, 0)),
        out_shape=jax.ShapeDtypeStruct((B, S, D), x.dtype),
        compiler_params=pltpu.CompilerParams(
            dimension_semantics=("parallel", "parallel"),
        ),
    )(x, pe)


def kernel(x, pe_weight):
    B, S, D = x.shape
    pe = pe_weight[:S]
    return _tc_add(x, pe)


# speedup vs baseline: 3.4401x; 1.0503x over previous
import functools

import jax
import jax.numpy as jnp
from jax.experimental import pallas as pl
from jax.experimental.pallas import tpu as pltpu
from jax.experimental.pallas import tpu_sc as plsc

_BR = 8
_LANES = 16


def _sc_add(xf, pe):
    R, D = xf.shape
    S = pe.shape[0]
    mesh = plsc.VectorSubcoreMesh(core_axis_name="c", subcore_axis_name="s")

    @functools.partial(
        pl.kernel,
        out_type=jax.ShapeDtypeStruct((R, D), xf.dtype),
        mesh=mesh,
    )
    def k(x_hbm, pe_hbm, o_hbm):
        def body(x_vmem, pe_vmem, o_vmem):
            @pl.loop(0, _BR)
            def _row(r):
                @pl.loop(0, D, step=_LANES)
                def _col(c):
                    slc = (pl.ds(r, 1), pl.ds(c, _LANES))
                    o_vmem.at[*slc][...] = (
                        x_vmem.at[*slc][...] + pe_vmem.at[*slc][...]
                    )

        pltpu.emit_pipeline(
            body,
            grid=(R // _BR, 1),
            in_specs=[
                pl.BlockSpec((_BR, D), lambda i, j: (i, j)),
                pl.BlockSpec((_BR, D), lambda i, j: (i % (S // _BR), j)),
            ],
            out_specs=[pl.BlockSpec((_BR, D), lambda i, j: (i, j))],
            core_axis_name=("c", "s"),
            dimension_semantics=(pltpu.PARALLEL, pltpu.PARALLEL),
        )(x_hbm, pe_hbm, o_hbm)

    return k(xf, pe)


_BS = 2048


def _tc_body(x_ref, pe_ref, o_ref):
    o_ref[...] = x_ref[...] + pe_ref[...]


def _tc_add(x, pe):
    B, S, D = x.shape
    grid = (S // _BS, B)
    return pl.pallas_call(
        _tc_body,
        grid=grid,
        in_specs=[
            pl.BlockSpec((1, _BS, D), lambda i, b: (b, i, 0)),
            pl.BlockSpec((_BS, D), lambda i, b: (i, 0)),
        ],
        out_specs=pl.BlockSpec((1, _BS, D), lambda i, b: (b, i, 0)),
        out_shape=jax.ShapeDtypeStruct((B, S, D), x.dtype),
        compiler_params=pltpu.CompilerParams(
            dimension_semantics=("parallel", "parallel"),
        ),
    )(x, pe)


def kernel(x, pe_weight):
    B, S, D = x.shape
    pe = pe_weight[:S]
    return _tc_add(x, pe)
